# baseline (speedup 1.0000x reference)
import functools

import jax
import jax.numpy as jnp
from jax.experimental import pallas as pl
from jax.experimental.pallas import tpu as pltpu

_BN_EPS = 1e-5
_NEG_SLOPE = 0.01


def _fused_body(x_ref, w_ref, g_ref, b_ref, o_ref,
                y_scr, sum_scr, ssq_scr, a_scr, bias_scr, *, m, hb, h_blk):
    p = pl.program_id(0)
    s = pl.program_id(1)
    n = s // hb
    k = s % hb

    @pl.when(p == 0)
    def _stats_phase():
        @pl.when(s == 0)
        def _():
            sum_scr[...] = jnp.zeros_like(sum_scr)
            ssq_scr[...] = jnp.zeros_like(ssq_scr)

        y = jax.lax.dot_general(
            w_ref[...], x_ref[0], (((1,), (0,)), ((), ())),
            preferred_element_type=jnp.float32)
        y_scr[n, :, pl.ds(k * h_blk, h_blk), :] = y.astype(y_scr.dtype)
        sum_scr[...] += jnp.sum(y, axis=1)
        ssq_scr[...] += jnp.sum(y * y, axis=1)

    @pl.when(p == 1)
    def _apply_phase():
        @pl.when(s == 0)
        def _():
            inv_m = 1.0 / m
            mean = jnp.sum(sum_scr[...], axis=1, keepdims=True) * inv_m
            ey2 = jnp.sum(ssq_scr[...], axis=1, keepdims=True) * inv_m
            var = jnp.maximum(ey2 - mean * mean, 0.0)
            a = g_ref[...] * jax.lax.rsqrt(var + _BN_EPS)
            a_scr[...] = a
            bias_scr[...] = b_ref[...] - a * mean

        y = y_scr[n, :, pl.ds(k * h_blk, h_blk), :].astype(jnp.float32)
        z = y * a_scr[...][:, :, None] + bias_scr[...][:, :, None]
        o_ref[0] = jnp.maximum(z, _NEG_SLOPE * z).astype(o_ref.dtype)


@jax.jit
def _conv_bn_leaky(x_nchw, conv_w, gamma, beta):
    N, Cin, H, W = x_nchw.shape
    Cout = conv_w.shape[0]
    m = float(N * H * W)
    h_blk = 32 if H % 32 == 0 else H
    hb = H // h_blk
    steps = N * hb

    w2 = conv_w.reshape(Cout, Cin)
    g1 = gamma.astype(jnp.float32).reshape(Cout, 1)
    b1 = beta.astype(jnp.float32).reshape(Cout, 1)

    body = functools.partial(_fused_body, m=m, hb=hb, h_blk=h_blk)
    return pl.pallas_call(
        body,
        out_shape=jax.ShapeDtypeStruct((N, Cout, H, W), x_nchw.dtype),
        grid=(2, steps),
        in_specs=[
            pl.BlockSpec(
                (1, Cin, h_blk, W),
                lambda p, s: ((s // hb) * (1 - p) + (N - 1) * p, 0,
                              (s % hb) * (1 - p) + (hb - 1) * p, 0)),
            pl.BlockSpec((Cout, Cin), lambda p, s: (0, 0)),
            pl.BlockSpec((Cout, 1), lambda p, s: (0, 0)),
            pl.BlockSpec((Cout, 1), lambda p, s: (0, 0)),
        ],
        out_specs=pl.BlockSpec(
            (1, Cout, h_blk, W),
            lambda p, s: (p * (s // hb), 0, p * (s % hb), 0)),
        scratch_shapes=[
            pltpu.VMEM((N, Cout, H, W), jnp.bfloat16),
            pltpu.VMEM((Cout, W), jnp.float32),
            pltpu.VMEM((Cout, W), jnp.float32),
            pltpu.VMEM((Cout, 1), jnp.float32),
            pltpu.VMEM((Cout, 1), jnp.float32),
        ],
        compiler_params=pltpu.CompilerParams(
            dimension_semantics=("arbitrary", "arbitrary"),
            vmem_limit_bytes=58 * 1024 * 1024),
    )(x_nchw, w2, g1, b1)


def kernel(x_nchw, conv_w, gamma, beta):
    return [_conv_bn_leaky(x_nchw, conv_w, gamma, beta)]


# speedup vs baseline: 1.1557x; 1.1557x over previous
import functools

import jax
import jax.numpy as jnp
from jax.experimental import pallas as pl
from jax.experimental.pallas import tpu as pltpu

_BN_EPS = 1e-5
_NEG_SLOPE = 0.01


def _fused_body(x_ref, w_ref, g_ref, b_ref, o_ref,
                xflat, gram, sumx, wp_scr, bias_scr, *, m, hb, h_blk, w_dim):
    p = pl.program_id(0)
    s = pl.program_id(1)
    n = s // hb
    k = s % hb
    t = h_blk * w_dim

    @pl.when(p == 0)
    def _stats_phase():
        @pl.when(s == 0)
        def _():
            sumx[...] = jnp.zeros_like(sumx)
            gram[...] = jnp.zeros_like(gram)

        xf = x_ref[0].reshape(x_ref.shape[1], t)
        xflat[n, :, pl.ds(k * t, t)] = xf
        sumx[...] += jnp.sum(xf, axis=1, keepdims=True)
        gram[...] += jax.lax.dot_general(
            xf, xf, (((1,), (1,)), ((), ())),
            preferred_element_type=jnp.float32)

    @pl.when(p == 1)
    def _apply_phase():
        @pl.when(s == 0)
        def _():
            inv_m = 1.0 / m
            w_f = w_ref[...]
            sum_y = jnp.dot(w_f, sumx[...],
                            preferred_element_type=jnp.float32)
            mean = sum_y * inv_m
            wg = jnp.dot(w_f, gram[...],
                         preferred_element_type=jnp.float32)
            sum_y2 = jnp.sum(wg * w_f, axis=1, keepdims=True)
            var = jnp.maximum(sum_y2 * inv_m - mean * mean, 0.0)
            a = g_ref[...] * jax.lax.rsqrt(var + _BN_EPS)
            wp_scr[...] = w_f * a
            bias_scr[...] = b_ref[...] - a * mean

        xf = xflat[n, :, pl.ds(k * t, t)]
        y = jnp.dot(wp_scr[...], xf, preferred_element_type=jnp.float32)
        z = y + bias_scr[...]
        z = jnp.maximum(z, _NEG_SLOPE * z)
        o_ref[0] = z.reshape(o_ref.shape[1], h_blk, w_dim)


@jax.jit
def _conv_bn_leaky(x_nchw, conv_w, gamma, beta):
    N, Cin, H, W = x_nchw.shape
    Cout = conv_w.shape[0]
    m = float(N * H * W)
    h_blk = 32 if H % 32 == 0 else H
    hb = H // h_blk
    steps = N * hb

    w2 = conv_w.reshape(Cout, Cin)
    g1 = gamma.astype(jnp.float32).reshape(Cout, 1)
    b1 = beta.astype(jnp.float32).reshape(Cout, 1)

    body = functools.partial(_fused_body, m=m, hb=hb, h_blk=h_blk, w_dim=W)
    return pl.pallas_call(
        body,
        out_shape=jax.ShapeDtypeStruct((N, Cout, H, W), x_nchw.dtype),
        grid=(2, steps),
        in_specs=[
            pl.BlockSpec(
                (1, Cin, h_blk, W),
                lambda p, s: ((s // hb) * (1 - p) + (N - 1) * p, 0,
                              (s % hb) * (1 - p) + (hb - 1) * p, 0)),
            pl.BlockSpec((Cout, Cin), lambda p, s: (0, 0)),
            pl.BlockSpec((Cout, 1), lambda p, s: (0, 0)),
            pl.BlockSpec((Cout, 1), lambda p, s: (0, 0)),
        ],
        out_specs=pl.BlockSpec(
            (1, Cout, h_blk, W),
            lambda p, s: (p * (s // hb), 0, p * (s % hb), 0)),
        scratch_shapes=[
            pltpu.VMEM((N, Cin, H * W), jnp.float32),
            pltpu.VMEM((Cin, Cin), jnp.float32),
            pltpu.VMEM((Cin, 1), jnp.float32),
            pltpu.VMEM((Cout, Cin), jnp.float32),
            pltpu.VMEM((Cout, 1), jnp.float32),
        ],
        compiler_params=pltpu.CompilerParams(
            dimension_semantics=("arbitrary", "arbitrary"),
            vmem_limit_bytes=58 * 1024 * 1024),
    )(x_nchw, w2, g1, b1)


def kernel(x_nchw, conv_w, gamma, beta):
    return [_conv_bn_leaky(x_nchw, conv_w, gamma, beta)]


# speedup vs baseline: 7.3162x; 6.3306x over previous
import functools

import jax
import jax.numpy as jnp
from jax.experimental import pallas as pl
from jax.experimental.pallas import tpu as pltpu

_BN_EPS = 1e-5
_NEG_SLOPE = 0.01


def _fused_body(x_ref, w_ref, g_ref, b_ref, o_ref,
                y_scr, sum_scr, ssq_scr, a_scr, bias_scr, *, m):
    p = pl.program_id(0)
    n = pl.program_id(1)
    hw = x_ref.shape[1] * x_ref.shape[2]
    cin = x_ref.shape[3]

    @pl.when(p == 0)
    def _stats_phase():
        @pl.when(n == 0)
        def _():
            sum_scr[...] = jnp.zeros_like(sum_scr)
            ssq_scr[...] = jnp.zeros_like(ssq_scr)

        x2 = x_ref[0].reshape(hw, cin)
        y = jax.lax.dot_general(
            x2, w_ref[...], (((1,), (1,)), ((), ())),
            preferred_element_type=jnp.float32)
        y_scr[n] = y
        sum_scr[...] += jnp.sum(y, axis=0, keepdims=True)
        ssq_scr[...] += jnp.sum(y * y, axis=0, keepdims=True)

    @pl.when(p == 1)
    def _apply_phase():
        @pl.when(n == 0)
        def _():
            inv_m = 1.0 / m
            mean = sum_scr[...] * inv_m
            var = jnp.maximum(ssq_scr[...] * inv_m - mean * mean, 0.0)
            a = g_ref[...] * jax.lax.rsqrt(var + _BN_EPS)
            a_scr[...] = a
            bias_scr[...] = b_ref[...] - a * mean

        z = y_scr[n] * a_scr[...] + bias_scr[...]
        z = jnp.maximum(z, _NEG_SLOPE * z)
        o_ref[0] = z.reshape(o_ref.shape[1], o_ref.shape[2], o_ref.shape[3])


@jax.jit
def _conv_bn_leaky(x_nchw, conv_w, gamma, beta):
    N, Cin, H, W = x_nchw.shape
    Cout = conv_w.shape[0]
    m = float(N * H * W)

    xt = jnp.transpose(x_nchw, (0, 2, 3, 1))
    w2 = conv_w.reshape(Cout, Cin)
    g1 = gamma.astype(jnp.float32).reshape(1, Cout)
    b1 = beta.astype(jnp.float32).reshape(1, Cout)

    body = functools.partial(_fused_body, m=m)
    out_t = pl.pallas_call(
        body,
        out_shape=jax.ShapeDtypeStruct((N, H, W, Cout), x_nchw.dtype),
        grid=(2, N),
        in_specs=[
            pl.BlockSpec((1, H, W, Cin),
                         lambda p, n: (n * (1 - p) + (N - 1) * p, 0, 0, 0)),
            pl.BlockSpec((Cout, Cin), lambda p, n: (0, 0)),
            pl.BlockSpec((1, Cout), lambda p, n: (0, 0)),
            pl.BlockSpec((1, Cout), lambda p, n: (0, 0)),
        ],
        out_specs=pl.BlockSpec((1, H, W, Cout),
                               lambda p, n: (n * p, 0, 0, 0)),
        scratch_shapes=[
            pltpu.VMEM((N, H * W, Cout), jnp.float32),
            pltpu.VMEM((1, Cout), jnp.float32),
            pltpu.VMEM((1, Cout), jnp.float32),
            pltpu.VMEM((1, Cout), jnp.float32),
            pltpu.VMEM((1, Cout), jnp.float32),
        ],
        compiler_params=pltpu.CompilerParams(
            dimension_semantics=("arbitrary", "arbitrary"),
            vmem_limit_bytes=58 * 1024 * 1024),
    )(xt, w2, g1, b1)

    return jnp.transpose(out_t, (0, 3, 1, 2))


def kernel(x_nchw, conv_w, gamma, beta):
    return [_conv_bn_leaky(x_nchw, conv_w, gamma, beta)]


# speedup vs baseline: 7.7522x; 1.0596x over previous
import functools

import jax
import jax.numpy as jnp
from jax.experimental import pallas as pl
from jax.experimental.pallas import tpu as pltpu

_BN_EPS = 1e-5
_NEG_SLOPE = 0.01


def _fused_body(x_ref, w_ref, g_ref, b_ref, o_ref,
                y_scr, wt_scr, sum_scr, ssq_scr, a_scr, bias_scr, *, m):
    p = pl.program_id(0)
    n = pl.program_id(1)
    hw = x_ref.shape[1] * x_ref.shape[2]
    cin = x_ref.shape[3]
    cout = wt_scr.shape[1]

    @pl.when(p == 0)
    def _stats_phase():
        @pl.when(n == 0)
        def _():
            wt_scr[...] = w_ref[...].reshape(cout, cin).T
            sum_scr[...] = jnp.zeros_like(sum_scr)
            ssq_scr[...] = jnp.zeros_like(ssq_scr)

        x2 = x_ref[0].reshape(hw, cin)
        y = jax.lax.dot_general(
            x2, wt_scr[...], (((1,), (0,)), ((), ())),
            preferred_element_type=jnp.float32)
        y_scr[n] = y
        sum_scr[...] += jnp.sum(y, axis=0, keepdims=True)
        ssq_scr[...] += jnp.sum(y * y, axis=0, keepdims=True)

    @pl.when(p == 1)
    def _apply_phase():
        @pl.when(n == 0)
        def _():
            inv_m = 1.0 / m
            mean = sum_scr[...] * inv_m
            var = jnp.maximum(ssq_scr[...] * inv_m - mean * mean, 0.0)
            a = g_ref[...] * jax.lax.rsqrt(var + _BN_EPS)
            a_scr[...] = a
            bias_scr[...] = b_ref[...] - a * mean

        z = y_scr[n] * a_scr[...] + bias_scr[...]
        z = jnp.maximum(z, _NEG_SLOPE * z)
        o_ref[0] = z.reshape(o_ref.shape[1], o_ref.shape[2], o_ref.shape[3])


@jax.jit
def _conv_bn_leaky(x_nchw, conv_w, gamma, beta):
    N, Cin, H, W = x_nchw.shape
    Cout = conv_w.shape[0]
    m = float(N * H * W)

    xt = jnp.transpose(x_nchw, (0, 2, 3, 1))
    w_r = conv_w.reshape(Cout * Cin // 128, 128)
    g1 = gamma.astype(jnp.float32).reshape(1, Cout)
    b1 = beta.astype(jnp.float32).reshape(1, Cout)

    body = functools.partial(_fused_body, m=m)
    out_t = pl.pallas_call(
        body,
        out_shape=jax.ShapeDtypeStruct((N, H, W, Cout), x_nchw.dtype),
        grid=(2, N),
        in_specs=[
            pl.BlockSpec((1, H, W, Cin),
                         lambda p, n: (n * (1 - p) + (N - 1) * p, 0, 0, 0)),
            pl.BlockSpec((Cout * Cin // 128, 128), lambda p, n: (0, 0)),
            pl.BlockSpec((1, Cout), lambda p, n: (0, 0)),
            pl.BlockSpec((1, Cout), lambda p, n: (0, 0)),
        ],
        out_specs=pl.BlockSpec((1, H, W, Cout),
                               lambda p, n: (n * p, 0, 0, 0)),
        scratch_shapes=[
            pltpu.VMEM((N, H * W, Cout), jnp.float32),
            pltpu.VMEM((Cin, Cout), jnp.float32),
            pltpu.VMEM((1, Cout), jnp.float32),
            pltpu.VMEM((1, Cout), jnp.float32),
            pltpu.VMEM((1, Cout), jnp.float32),
            pltpu.VMEM((1, Cout), jnp.float32),
        ],
        compiler_params=pltpu.CompilerParams(
            dimension_semantics=("arbitrary", "arbitrary"),
            vmem_limit_bytes=58 * 1024 * 1024),
    )(xt, w_r, g1, b1)

    return jnp.transpose(out_t, (0, 3, 1, 2))


def kernel(x_nchw, conv_w, gamma, beta):
    return [_conv_bn_leaky(x_nchw, conv_w, gamma, beta)]
